# baseline (speedup 1.0000x reference)
import functools

import jax
import jax.numpy as jnp
from jax import lax
from jax.experimental import pallas as pl
from jax.experimental.pallas import tpu as pltpu
from jax.experimental.pallas import tpu_sc as plsc

NC = 2
NS = 16
NW = NC * NS
CH = 128
F = 16
NSLOT = 4

NCH0 = 40
NCH1 = 120
NCHM = max(NCH0, NCH1)
G0 = NCH0 // NSLOT
G1 = NCH1 // NSLOT


def _chunk_layout(c, s):
    nch = jnp.where(c == 0, NCH0, NCH1)
    base = jnp.where(c == 0, s * NCH0, NS * NCH0 + s * NCH1)
    ngrp = jnp.where(c == 0, G0, G1)
    return nch, base, ngrp


def _sc_agg(NP, rows_per_tile):
    mesh = plsc.VectorSubcoreMesh(core_axis_name="c", subcore_axis_name="s")

    @functools.partial(
        pl.kernel,
        out_type=jax.ShapeDtypeStruct((NC, NP, F), jnp.float32),
        mesh=mesh,
        scratch_types=[
            pltpu.VMEM((NCHM, CH), jnp.int32),
            pltpu.VMEM((NCHM, CH), jnp.int32),
            [pltpu.VMEM((CH, F), jnp.float32) for _ in range(NSLOT)],
            pltpu.VMEM((rows_per_tile, F), jnp.float32),
            pltpu.VMEM_SHARED((NP, F), jnp.float32),
            [pltpu.SemaphoreType.DMA for _ in range(NSLOT)],
            [pltpu.SemaphoreType.DMA for _ in range(NSLOT)],
        ],
        compiler_params=pltpu.CompilerParams(use_tc_tiling_on_sc=False),
    )
    def agg(y_hbm, src_hbm, dst_hbm, out_hbm, sidx, didx, bufs,
            stage, acc, gsems, ssems):
        c = lax.axis_index("c")
        s = lax.axis_index("s")
        base = s * rows_per_tile
        nch, chunk_base, ngrp = _chunk_layout(c, s)

        zrow = jnp.zeros((F,), jnp.float32)

        def zero_stage(i, carry):
            stage[i, :] = zrow
            return carry

        lax.fori_loop(0, rows_per_tile, zero_stage, 0)

        pltpu.sync_copy(stage, acc.at[pl.ds(base, rows_per_tile)])

        pltpu.sync_copy(src_hbm.at[pl.ds(chunk_base, NCHM)], sidx)
        pltpu.sync_copy(dst_hbm.at[pl.ds(chunk_base, NCHM)], didx)
        plsc.subcore_barrier()

        def fire_gather(j, b):
            pltpu.async_copy(y_hbm.at[sidx.at[j]], bufs[b], gsems[b])

        def drain(sem, b):
            pltpu.make_async_copy(y_hbm.at[pl.ds(0, CH)], bufs[b], sem).wait()

        for b in range(NSLOT):
            fire_gather(b, b)

        def group(g, carry):
            for b in range(NSLOT):
                j = g * NSLOT + b
                drain(gsems[b], b)
                pltpu.async_copy(bufs[b], acc.at[didx.at[j]], ssems[b],
                                 add=True)
            for b in range(NSLOT):
                drain(ssems[b], b)

                @pl.when(g < ngrp - 1)
                def _():
                    fire_gather((g + 1) * NSLOT + b, b)

            return carry

        lax.fori_loop(0, ngrp, group, 0)
        plsc.subcore_barrier()

        pltpu.sync_copy(acc.at[pl.ds(base, rows_per_tile)], stage)
        pltpu.sync_copy(stage, out_hbm.at[c].at[pl.ds(base, rows_per_tile)])

    return agg


def _sc_deg(NP, rows_per_tile):
    mesh = plsc.VectorSubcoreMesh(core_axis_name="c", subcore_axis_name="s")

    @functools.partial(
        pl.kernel,
        out_type=jax.ShapeDtypeStruct((NC, NP), jnp.float32),
        mesh=mesh,
        scratch_types=[
            pltpu.VMEM((NCHM, CH), jnp.int32),
            pltpu.VMEM((CH,), jnp.float32),
            pltpu.VMEM((rows_per_tile,), jnp.float32),
            pltpu.VMEM_SHARED((NP,), jnp.float32),
            pltpu.SemaphoreType.DMA,
        ],
        compiler_params=pltpu.CompilerParams(use_tc_tiling_on_sc=False),
    )
    def deg(dst_hbm, out_hbm, didx, ones, stage, acc, sem):
        c = lax.axis_index("c")
        s = lax.axis_index("s")
        base = s * rows_per_tile
        nch, chunk_base, _ = _chunk_layout(c, s)

        zv = jnp.zeros((F,), jnp.float32)
        ov = jnp.ones((F,), jnp.float32)

        def zero_stage(i, carry):
            stage[pl.ds(i * F, F)] = zv
            return carry

        lax.fori_loop(0, rows_per_tile // F, zero_stage, 0)

        def fill_ones(i, carry):
            ones[pl.ds(i * F, F)] = ov
            return carry

        lax.fori_loop(0, CH // F, fill_ones, 0)

        pltpu.sync_copy(stage, acc.at[pl.ds(base, rows_per_tile)])
        pltpu.sync_copy(dst_hbm.at[pl.ds(chunk_base, NCHM)], didx)
        plsc.subcore_barrier()

        def chunk(j, carry):
            pltpu.async_copy(ones, acc.at[didx.at[j]], sem, add=True)
            return carry

        lax.fori_loop(0, nch, chunk, 0)

        def undrain(j, carry):
            pltpu.make_async_copy(dst_hbm.at[0], didx.at[0], sem).wait()
            return carry

        lax.fori_loop(0, nch, undrain, 0)
        plsc.subcore_barrier()

        pltpu.sync_copy(acc.at[pl.ds(base, rows_per_tile)], stage)
        pltpu.sync_copy(stage, out_hbm.at[c].at[pl.ds(base, rows_per_tile)])

    return deg


def _tc_stage1(N, R):
    def body(x_ref, w1_ref, dp_ref, y1_ref, dinv_ref):
        deg = 1.0 + dp_ref[0] + dp_ref[1]
        dinv = lax.rsqrt(deg)
        xw = jnp.dot(
            x_ref[...], w1_ref[...],
            preferred_element_type=jnp.float32,
            precision=lax.Precision.HIGHEST,
        )
        dinv_ref[...] = dinv
        y1_ref[...] = dinv * xw

    grid = (N // R,)
    return pl.pallas_call(
        body,
        grid=grid,
        in_specs=[
            pl.BlockSpec((R, 128), lambda i: (i, 0)),
            pl.BlockSpec((128, F), lambda i: (0, 0)),
            pl.BlockSpec((NC, R, 1), lambda i: (0, i, 0)),
        ],
        out_specs=[
            pl.BlockSpec((R, F), lambda i: (i, 0)),
            pl.BlockSpec((R, 1), lambda i: (i, 0)),
        ],
        out_shape=[
            jax.ShapeDtypeStruct((N, F), jnp.float32),
            jax.ShapeDtypeStruct((N, 1), jnp.float32),
        ],
    )


def _tc_stage2(N, R):
    def body(p_ref, y1_ref, dinv_ref, b1_ref, y2_ref):
        full = y1_ref[...] + p_ref[0] + p_ref[1]
        dinv = dinv_ref[...]
        h = jnp.maximum(dinv * full + b1_ref[...], 0.0)
        y2_ref[...] = dinv * h

    grid = (N // R,)
    return pl.pallas_call(
        body,
        grid=grid,
        in_specs=[
            pl.BlockSpec((NC, R, F), lambda i: (0, i, 0)),
            pl.BlockSpec((R, F), lambda i: (i, 0)),
            pl.BlockSpec((R, 1), lambda i: (i, 0)),
            pl.BlockSpec((1, F), lambda i: (0, 0)),
        ],
        out_specs=pl.BlockSpec((R, F), lambda i: (i, 0)),
        out_shape=jax.ShapeDtypeStruct((N, F), jnp.float32),
    )


def _tc_stage3(N, R, d_out):
    def body(q_ref, y2_ref, dinv_ref, w2_ref, b2_ref, o_ref):
        full = y2_ref[...] + q_ref[0] + q_ref[1]
        z = dinv_ref[...] * full
        o_ref[...] = (
            jnp.dot(
                z, w2_ref[...],
                preferred_element_type=jnp.float32,
                precision=lax.Precision.HIGHEST,
            )
            + b2_ref[...]
        )

    grid = (N // R,)
    return pl.pallas_call(
        body,
        grid=grid,
        in_specs=[
            pl.BlockSpec((NC, R, F), lambda i: (0, i, 0)),
            pl.BlockSpec((R, F), lambda i: (i, 0)),
            pl.BlockSpec((R, 1), lambda i: (i, 0)),
            pl.BlockSpec((F, d_out), lambda i: (0, 0)),
            pl.BlockSpec((1, d_out), lambda i: (0, 0)),
        ],
        out_specs=pl.BlockSpec((R, d_out), lambda i: (i, 0)),
        out_shape=jax.ShapeDtypeStruct((N, d_out), jnp.float32),
    )


def kernel(x, edge_index, W1, b1, W2, b2):
    N = x.shape[0]
    d_out = W2.shape[1]
    E = edge_index.shape[1]

    tot_ch = NS * (NCH0 + NCH1)
    assert tot_ch * CH >= E
    e_pad = (tot_ch + NCHM) * CH - E
    src = edge_index[0].astype(jnp.int32)
    dst = edge_index[1].astype(jnp.int32)
    src = jnp.concatenate([src, jnp.zeros((e_pad,), jnp.int32)])
    dst = jnp.concatenate([dst, jnp.full((e_pad,), N, jnp.int32)])
    src = src.reshape(tot_ch + NCHM, CH)
    dst = dst.reshape(tot_ch + NCHM, CH)

    rows_per_tile = -(-N // (NS * 8)) * 8
    NP = NS * rows_per_tile

    deg_k = _sc_deg(NP, rows_per_tile)
    agg_k = _sc_agg(NP, rows_per_tile)
    R = 2000
    tc1 = _tc_stage1(N, R)
    tc2 = _tc_stage2(N, R)
    tc3 = _tc_stage3(N, R, d_out)

    degp = deg_k(dst).reshape(NC, NP, 1)
    y1, dinv = tc1(x, W1, degp)
    p = agg_k(y1, src, dst)
    y2 = tc2(p, y1, dinv, b1.reshape(1, F))
    q = agg_k(y2, src, dst)
    out = tc3(q, y2, dinv, W2, b2.reshape(1, d_out))
    return out


# speedup vs baseline: 1.0810x; 1.0810x over previous
import functools

import jax
import jax.numpy as jnp
from jax import lax
from jax.experimental import pallas as pl
from jax.experimental.pallas import tpu as pltpu
from jax.experimental.pallas import tpu_sc as plsc

NC = 2
NS = 16
NW = NC * NS
CH = 128
F = 16
NSLOT = 4

NCH0 = 80
NCH1 = 80
NCHM = max(NCH0, NCH1)
G0 = NCH0 // NSLOT
G1 = NCH1 // NSLOT


def _chunk_layout(c, s):
    nch = jnp.where(c == 0, NCH0, NCH1)
    base = jnp.where(c == 0, s * NCH0, NS * NCH0 + s * NCH1)
    ngrp = jnp.where(c == 0, G0, G1)
    return nch, base, ngrp


def _sc_agg(NP, rows_per_tile):
    mesh = plsc.VectorSubcoreMesh(core_axis_name="c", subcore_axis_name="s")

    @functools.partial(
        pl.kernel,
        out_type=jax.ShapeDtypeStruct((NC, NP, F), jnp.float32),
        mesh=mesh,
        scratch_types=[
            pltpu.VMEM((NCHM, CH), jnp.int32),
            pltpu.VMEM((NCHM, CH), jnp.int32),
            [pltpu.VMEM((CH, F), jnp.float32) for _ in range(NSLOT)],
            pltpu.VMEM((rows_per_tile, F), jnp.float32),
            pltpu.VMEM_SHARED((NP, F), jnp.float32),
            [pltpu.SemaphoreType.DMA for _ in range(NSLOT)],
            [pltpu.SemaphoreType.DMA for _ in range(NSLOT)],
        ],
        compiler_params=pltpu.CompilerParams(use_tc_tiling_on_sc=False),
    )
    def agg(y_hbm, src_hbm, dst_hbm, out_hbm, sidx, didx, bufs,
            stage, acc, gsems, ssems):
        c = lax.axis_index("c")
        s = lax.axis_index("s")
        base = s * rows_per_tile
        nch, chunk_base, ngrp = _chunk_layout(c, s)

        zrow = jnp.zeros((F,), jnp.float32)

        def zero_stage(i, carry):
            stage[i, :] = zrow
            return carry

        lax.fori_loop(0, rows_per_tile, zero_stage, 0)

        pltpu.sync_copy(stage, acc.at[pl.ds(base, rows_per_tile)])

        pltpu.sync_copy(src_hbm.at[pl.ds(chunk_base, NCHM)], sidx)
        pltpu.sync_copy(dst_hbm.at[pl.ds(chunk_base, NCHM)], didx)
        plsc.subcore_barrier()

        def fire_gather(j, b):
            pltpu.async_copy(y_hbm.at[sidx.at[j]], bufs[b], gsems[b])

        def drain(sem, b):
            pltpu.make_async_copy(y_hbm.at[pl.ds(0, CH)], bufs[b], sem).wait()

        for b in range(NSLOT):
            fire_gather(b, b)

        def group(g, carry):
            for b in range(NSLOT):
                j = g * NSLOT + b
                drain(gsems[b], b)
                pltpu.async_copy(bufs[b], acc.at[didx.at[j]], ssems[b],
                                 add=True)
            for b in range(NSLOT):
                drain(ssems[b], b)

                @pl.when(g < ngrp - 1)
                def _():
                    fire_gather((g + 1) * NSLOT + b, b)

            return carry

        lax.fori_loop(0, ngrp, group, 0)
        plsc.subcore_barrier()

        pltpu.sync_copy(acc.at[pl.ds(base, rows_per_tile)], stage)
        pltpu.sync_copy(stage, out_hbm.at[c].at[pl.ds(base, rows_per_tile)])

    return agg


def _sc_deg(NP, rows_per_tile):
    mesh = plsc.VectorSubcoreMesh(core_axis_name="c", subcore_axis_name="s")

    @functools.partial(
        pl.kernel,
        out_type=jax.ShapeDtypeStruct((NC, NP), jnp.float32),
        mesh=mesh,
        scratch_types=[
            pltpu.VMEM((NCHM, CH), jnp.int32),
            pltpu.VMEM((CH,), jnp.float32),
            pltpu.VMEM((rows_per_tile,), jnp.float32),
            pltpu.VMEM_SHARED((NP,), jnp.float32),
            pltpu.SemaphoreType.DMA,
        ],
        compiler_params=pltpu.CompilerParams(use_tc_tiling_on_sc=False),
    )
    def deg(dst_hbm, out_hbm, didx, ones, stage, acc, sem):
        c = lax.axis_index("c")
        s = lax.axis_index("s")
        base = s * rows_per_tile
        nch, chunk_base, _ = _chunk_layout(c, s)

        zv = jnp.zeros((F,), jnp.float32)
        ov = jnp.ones((F,), jnp.float32)

        def zero_stage(i, carry):
            stage[pl.ds(i * F, F)] = zv
            return carry

        lax.fori_loop(0, rows_per_tile // F, zero_stage, 0)

        def fill_ones(i, carry):
            ones[pl.ds(i * F, F)] = ov
            return carry

        lax.fori_loop(0, CH // F, fill_ones, 0)

        pltpu.sync_copy(stage, acc.at[pl.ds(base, rows_per_tile)])
        pltpu.sync_copy(dst_hbm.at[pl.ds(chunk_base, NCHM)], didx)
        plsc.subcore_barrier()

        def chunk(j, carry):
            pltpu.async_copy(ones, acc.at[didx.at[j]], sem, add=True)
            return carry

        lax.fori_loop(0, nch, chunk, 0)

        def undrain(j, carry):
            pltpu.make_async_copy(dst_hbm.at[0], didx.at[0], sem).wait()
            return carry

        lax.fori_loop(0, nch, undrain, 0)
        plsc.subcore_barrier()

        pltpu.sync_copy(acc.at[pl.ds(base, rows_per_tile)], stage)
        pltpu.sync_copy(stage, out_hbm.at[c].at[pl.ds(base, rows_per_tile)])

    return deg


def _tc_stage1(N, R):
    def body(x_ref, w1_ref, dp_ref, y1_ref, dinv_ref):
        deg = 1.0 + dp_ref[0] + dp_ref[1]
        dinv = lax.rsqrt(deg)
        xw = jnp.dot(
            x_ref[...], w1_ref[...],
            preferred_element_type=jnp.float32,
            precision=lax.Precision.HIGHEST,
        )
        dinv_ref[...] = dinv
        y1_ref[...] = dinv * xw

    grid = (N // R,)
    return pl.pallas_call(
        body,
        grid=grid,
        in_specs=[
            pl.BlockSpec((R, 128), lambda i: (i, 0)),
            pl.BlockSpec((128, F), lambda i: (0, 0)),
            pl.BlockSpec((NC, R, 1), lambda i: (0, i, 0)),
        ],
        out_specs=[
            pl.BlockSpec((R, F), lambda i: (i, 0)),
            pl.BlockSpec((R, 1), lambda i: (i, 0)),
        ],
        out_shape=[
            jax.ShapeDtypeStruct((N, F), jnp.float32),
            jax.ShapeDtypeStruct((N, 1), jnp.float32),
        ],
    )


def _tc_stage2(N, R):
    def body(p_ref, y1_ref, dinv_ref, b1_ref, y2_ref):
        full = y1_ref[...] + p_ref[0] + p_ref[1]
        dinv = dinv_ref[...]
        h = jnp.maximum(dinv * full + b1_ref[...], 0.0)
        y2_ref[...] = dinv * h

    grid = (N // R,)
    return pl.pallas_call(
        body,
        grid=grid,
        in_specs=[
            pl.BlockSpec((NC, R, F), lambda i: (0, i, 0)),
            pl.BlockSpec((R, F), lambda i: (i, 0)),
            pl.BlockSpec((R, 1), lambda i: (i, 0)),
            pl.BlockSpec((1, F), lambda i: (0, 0)),
        ],
        out_specs=pl.BlockSpec((R, F), lambda i: (i, 0)),
        out_shape=jax.ShapeDtypeStruct((N, F), jnp.float32),
    )


def _tc_stage3(N, R, d_out):
    def body(q_ref, y2_ref, dinv_ref, w2_ref, b2_ref, o_ref):
        full = y2_ref[...] + q_ref[0] + q_ref[1]
        z = dinv_ref[...] * full
        o_ref[...] = (
            jnp.dot(
                z, w2_ref[...],
                preferred_element_type=jnp.float32,
                precision=lax.Precision.HIGHEST,
            )
            + b2_ref[...]
        )

    grid = (N // R,)
    return pl.pallas_call(
        body,
        grid=grid,
        in_specs=[
            pl.BlockSpec((NC, R, F), lambda i: (0, i, 0)),
            pl.BlockSpec((R, F), lambda i: (i, 0)),
            pl.BlockSpec((R, 1), lambda i: (i, 0)),
            pl.BlockSpec((F, d_out), lambda i: (0, 0)),
            pl.BlockSpec((1, d_out), lambda i: (0, 0)),
        ],
        out_specs=pl.BlockSpec((R, d_out), lambda i: (i, 0)),
        out_shape=jax.ShapeDtypeStruct((N, d_out), jnp.float32),
    )


def kernel(x, edge_index, W1, b1, W2, b2):
    N = x.shape[0]
    d_out = W2.shape[1]
    E = edge_index.shape[1]

    rows_per_tile = -(-N // (NS * 8)) * 8
    NP = NS * rows_per_tile

    tot_ch = NS * (NCH0 + NCH1)
    assert tot_ch * CH >= E
    e_pad = (tot_ch + NCHM) * CH - E
    src = edge_index[0].astype(jnp.int32)
    dst = edge_index[1].astype(jnp.int32)
    pad_dst = N + jnp.arange(e_pad, dtype=jnp.int32) % (NP - N)
    src = jnp.concatenate([src, jnp.zeros((e_pad,), jnp.int32)])
    dst = jnp.concatenate([dst, pad_dst])
    src = src.reshape(tot_ch + NCHM, CH)
    dst = dst.reshape(tot_ch + NCHM, CH)

    deg_k = _sc_deg(NP, rows_per_tile)
    agg_k = _sc_agg(NP, rows_per_tile)
    R = 2000
    tc1 = _tc_stage1(N, R)
    tc2 = _tc_stage2(N, R)
    tc3 = _tc_stage3(N, R, d_out)

    degp = deg_k(dst).reshape(NC, NP, 1)
    y1, dinv = tc1(x, W1, degp)
    p = agg_k(y1, src, dst)
    y2 = tc2(p, y1, dinv, b1.reshape(1, F))
    q = agg_k(y2, src, dst)
    out = tc3(q, y2, dinv, W2, b2.reshape(1, d_out))
    return out


# speedup vs baseline: 1.6232x; 1.5015x over previous
import functools

import jax
import jax.numpy as jnp
from jax import lax
from jax.experimental import pallas as pl
from jax.experimental.pallas import tpu as pltpu
from jax.experimental.pallas import tpu_sc as plsc

NC = 2
NS = 16
NW = NC * NS
CH = 128
F = 16
NSLOT = 4

NCH0 = 80
NCH1 = 80
NCHM = max(NCH0, NCH1)
G0 = NCH0 // NSLOT
G1 = NCH1 // NSLOT


def _chunk_layout(c, s):
    nch = jnp.where(c == 0, NCH0, NCH1)
    base = jnp.where(c == 0, s * NCH0, NS * NCH0 + s * NCH1)
    ngrp = jnp.where(c == 0, G0, G1)
    return nch, base, ngrp


def _sc_agg(NP, rows_per_tile):
    mesh = plsc.VectorSubcoreMesh(core_axis_name="c", subcore_axis_name="s")

    @functools.partial(
        pl.kernel,
        out_type=jax.ShapeDtypeStruct((NC, NP, F), jnp.float32),
        mesh=mesh,
        scratch_types=[
            pltpu.VMEM((NCHM, CH), jnp.int32),
            pltpu.VMEM((NCHM, CH), jnp.int32),
            [pltpu.VMEM((CH, F), jnp.float32) for _ in range(NSLOT)],
            pltpu.VMEM((rows_per_tile, F), jnp.float32),
            pltpu.VMEM_SHARED((NP, F), jnp.float32),
            pltpu.VMEM_SHARED((NP, F), jnp.float32),
            [pltpu.SemaphoreType.DMA for _ in range(NSLOT)],
            [pltpu.SemaphoreType.DMA for _ in range(NSLOT)],
        ],
        compiler_params=pltpu.CompilerParams(use_tc_tiling_on_sc=False),
    )
    def agg(y_hbm, src_hbm, dst_hbm, out_hbm, sidx, didx, bufs,
            stage, acc, ysp, gsems, ssems):
        c = lax.axis_index("c")
        s = lax.axis_index("s")
        base = s * rows_per_tile
        N = y_hbm.shape[0]
        tail = N - (NS - 1) * rows_per_tile
        nch, chunk_base, ngrp = _chunk_layout(c, s)

        @pl.when(s < NS - 1)
        def _():
            pltpu.sync_copy(y_hbm.at[pl.ds(base, rows_per_tile)], stage)
            pltpu.sync_copy(stage, ysp.at[pl.ds(base, rows_per_tile)])

        @pl.when(s == NS - 1)
        def _():
            pltpu.sync_copy(y_hbm.at[pl.ds((NS - 1) * rows_per_tile, tail)],
                            stage.at[pl.ds(0, tail)])
            pltpu.sync_copy(stage.at[pl.ds(0, tail)],
                            ysp.at[pl.ds((NS - 1) * rows_per_tile, tail)])

        zrow = jnp.zeros((F,), jnp.float32)

        def zero_stage(i, carry):
            stage[i, :] = zrow
            return carry

        lax.fori_loop(0, rows_per_tile, zero_stage, 0)

        pltpu.sync_copy(stage, acc.at[pl.ds(base, rows_per_tile)])

        pltpu.sync_copy(src_hbm.at[pl.ds(chunk_base, NCHM)], sidx)
        pltpu.sync_copy(dst_hbm.at[pl.ds(chunk_base, NCHM)], didx)
        plsc.subcore_barrier()

        def fire_gather(j, b):
            pltpu.async_copy(ysp.at[sidx.at[j]], bufs[b], gsems[b])

        def drain(sem, b):
            pltpu.make_async_copy(y_hbm.at[pl.ds(0, CH)], bufs[b], sem).wait()

        for b in range(NSLOT):
            fire_gather(b, b)

        def group(g, carry):
            for b in range(NSLOT):
                j = g * NSLOT + b
                drain(gsems[b], b)
                pltpu.async_copy(bufs[b], acc.at[didx.at[j]], ssems[b],
                                 add=True)
            for b in range(NSLOT):
                drain(ssems[b], b)

                @pl.when(g < ngrp - 1)
                def _():
                    fire_gather((g + 1) * NSLOT + b, b)

            return carry

        lax.fori_loop(0, ngrp, group, 0)
        plsc.subcore_barrier()

        pltpu.sync_copy(acc.at[pl.ds(base, rows_per_tile)], stage)
        pltpu.sync_copy(stage, out_hbm.at[c].at[pl.ds(base, rows_per_tile)])

    return agg


def _sc_deg(NP, rows_per_tile):
    mesh = plsc.VectorSubcoreMesh(core_axis_name="c", subcore_axis_name="s")

    @functools.partial(
        pl.kernel,
        out_type=jax.ShapeDtypeStruct((NC, NP), jnp.float32),
        mesh=mesh,
        scratch_types=[
            pltpu.VMEM((NCHM, CH), jnp.int32),
            pltpu.VMEM((CH,), jnp.float32),
            pltpu.VMEM((rows_per_tile,), jnp.float32),
            pltpu.VMEM_SHARED((NP,), jnp.float32),
            pltpu.SemaphoreType.DMA,
        ],
        compiler_params=pltpu.CompilerParams(use_tc_tiling_on_sc=False),
    )
    def deg(dst_hbm, out_hbm, didx, ones, stage, acc, sem):
        c = lax.axis_index("c")
        s = lax.axis_index("s")
        base = s * rows_per_tile
        nch, chunk_base, _ = _chunk_layout(c, s)

        zv = jnp.zeros((F,), jnp.float32)
        ov = jnp.ones((F,), jnp.float32)

        def zero_stage(i, carry):
            stage[pl.ds(i * F, F)] = zv
            return carry

        lax.fori_loop(0, rows_per_tile // F, zero_stage, 0)

        def fill_ones(i, carry):
            ones[pl.ds(i * F, F)] = ov
            return carry

        lax.fori_loop(0, CH // F, fill_ones, 0)

        pltpu.sync_copy(stage, acc.at[pl.ds(base, rows_per_tile)])
        pltpu.sync_copy(dst_hbm.at[pl.ds(chunk_base, NCHM)], didx)
        plsc.subcore_barrier()

        def chunk(j, carry):
            pltpu.async_copy(ones, acc.at[didx.at[j]], sem, add=True)
            return carry

        lax.fori_loop(0, nch, chunk, 0)

        def undrain(j, carry):
            pltpu.make_async_copy(dst_hbm.at[0], didx.at[0], sem).wait()
            return carry

        lax.fori_loop(0, nch, undrain, 0)
        plsc.subcore_barrier()

        pltpu.sync_copy(acc.at[pl.ds(base, rows_per_tile)], stage)
        pltpu.sync_copy(stage, out_hbm.at[c].at[pl.ds(base, rows_per_tile)])

    return deg


def _tc_stage1(N, R):
    def body(x_ref, w1_ref, dp_ref, y1_ref, dinv_ref):
        deg = 1.0 + dp_ref[0] + dp_ref[1]
        dinv = lax.rsqrt(deg)
        xw = jnp.dot(
            x_ref[...], w1_ref[...],
            preferred_element_type=jnp.float32,
            precision=lax.Precision.HIGHEST,
        )
        dinv_ref[...] = dinv
        y1_ref[...] = dinv * xw

    grid = (N // R,)
    return pl.pallas_call(
        body,
        grid=grid,
        in_specs=[
            pl.BlockSpec((R, 128), lambda i: (i, 0)),
            pl.BlockSpec((128, F), lambda i: (0, 0)),
            pl.BlockSpec((NC, R, 1), lambda i: (0, i, 0)),
        ],
        out_specs=[
            pl.BlockSpec((R, F), lambda i: (i, 0)),
            pl.BlockSpec((R, 1), lambda i: (i, 0)),
        ],
        out_shape=[
            jax.ShapeDtypeStruct((N, F), jnp.float32),
            jax.ShapeDtypeStruct((N, 1), jnp.float32),
        ],
    )


def _tc_stage2(N, R):
    def body(p_ref, y1_ref, dinv_ref, b1_ref, y2_ref):
        full = y1_ref[...] + p_ref[0] + p_ref[1]
        dinv = dinv_ref[...]
        h = jnp.maximum(dinv * full + b1_ref[...], 0.0)
        y2_ref[...] = dinv * h

    grid = (N // R,)
    return pl.pallas_call(
        body,
        grid=grid,
        in_specs=[
            pl.BlockSpec((NC, R, F), lambda i: (0, i, 0)),
            pl.BlockSpec((R, F), lambda i: (i, 0)),
            pl.BlockSpec((R, 1), lambda i: (i, 0)),
            pl.BlockSpec((1, F), lambda i: (0, 0)),
        ],
        out_specs=pl.BlockSpec((R, F), lambda i: (i, 0)),
        out_shape=jax.ShapeDtypeStruct((N, F), jnp.float32),
    )


def _tc_stage3(N, R, d_out):
    def body(q_ref, y2_ref, dinv_ref, w2_ref, b2_ref, o_ref):
        full = y2_ref[...] + q_ref[0] + q_ref[1]
        z = dinv_ref[...] * full
        o_ref[...] = (
            jnp.dot(
                z, w2_ref[...],
                preferred_element_type=jnp.float32,
                precision=lax.Precision.HIGHEST,
            )
            + b2_ref[...]
        )

    grid = (N // R,)
    return pl.pallas_call(
        body,
        grid=grid,
        in_specs=[
            pl.BlockSpec((NC, R, F), lambda i: (0, i, 0)),
            pl.BlockSpec((R, F), lambda i: (i, 0)),
            pl.BlockSpec((R, 1), lambda i: (i, 0)),
            pl.BlockSpec((F, d_out), lambda i: (0, 0)),
            pl.BlockSpec((1, d_out), lambda i: (0, 0)),
        ],
        out_specs=pl.BlockSpec((R, d_out), lambda i: (i, 0)),
        out_shape=jax.ShapeDtypeStruct((N, d_out), jnp.float32),
    )


def kernel(x, edge_index, W1, b1, W2, b2):
    N = x.shape[0]
    d_out = W2.shape[1]
    E = edge_index.shape[1]

    rows_per_tile = -(-N // (NS * 8)) * 8
    NP = NS * rows_per_tile

    tot_ch = NS * (NCH0 + NCH1)
    assert tot_ch * CH >= E
    e_pad = (tot_ch + NCHM) * CH - E
    src = edge_index[0].astype(jnp.int32)
    dst = edge_index[1].astype(jnp.int32)
    pad_dst = N + jnp.arange(e_pad, dtype=jnp.int32) % (NP - N)
    src = jnp.concatenate([src, jnp.zeros((e_pad,), jnp.int32)])
    dst = jnp.concatenate([dst, pad_dst])
    src = src.reshape(tot_ch + NCHM, CH)
    dst = dst.reshape(tot_ch + NCHM, CH)

    deg_k = _sc_deg(NP, rows_per_tile)
    agg_k = _sc_agg(NP, rows_per_tile)
    R = 2000
    tc1 = _tc_stage1(N, R)
    tc2 = _tc_stage2(N, R)
    tc3 = _tc_stage3(N, R, d_out)

    degp = deg_k(dst).reshape(NC, NP, 1)
    y1, dinv = tc1(x, W1, degp)
    p = agg_k(y1, src, dst)
    y2 = tc2(p, y1, dinv, b1.reshape(1, F))
    q = agg_k(y2, src, dst)
    out = tc3(q, y2, dinv, W2, b2.reshape(1, d_out))
    return out
